# baseline (speedup 1.0000x reference)
import functools

import jax
import jax.numpy as jnp
from jax import lax
from jax.experimental import pallas as pl
from jax.experimental.pallas import tpu as pltpu
from jax.experimental.pallas import tpu_sc as plsc

N = 10000
D = 128
OUT = 40
NP = 10240
RPT = NP // 16
E = 320000
CH = 128
KCH = 80
EPT = CH * KCH
EP = EPT * 32
NCHUNK = EP // CH
DW = 128

_MESH = dict(core_axis_name="c", subcore_axis_name="s")

NIB = 8
NRB = 2
PF = 4
NSS = 4


def _mp_body(z_hbm, edges_hbm, zrow_hbm, out_hbm, *scr):
    ib = scr[0:NIB]
    rb = scr[NIB:NIB + NRB]
    acc_sh = scr[NIB + NRB]
    isem = scr[NIB + NRB + 1:NIB + NRB + 1 + NIB]
    gsem = scr[NIB + NRB + 1 + NIB:NIB + NRB + 1 + NIB + NRB]
    ssem = scr[NIB + NRB + 1 + NIB + NRB:]
    c = lax.axis_index("c")
    s = lax.axis_index("s")
    r0 = s * RPT
    pltpu.sync_copy(zrow_hbm, acc_sh.at[pl.ds(r0, RPT)])
    plsc.subcore_barrier()
    tb = (c * 16 + s) * KCH

    def idx_load(k, j):
        pltpu.async_copy(edges_hbm.at[tb + k], ib[j], isem[j])

    for j in range(PF):
        idx_load(j, j)

    def group(g, carry):
        for j in range(NIB):
            k = g * NIB + j
            rj = j % NRB
            pj = (j - 1) % NRB
            ij = (j - 1) % NIB

            @pl.when(k >= NRB)
            def _():
                pltpu.make_async_copy(
                    rb[rj], acc_sh.at[ib[(j - NRB) % NIB].at[1]],
                    ssem[rj]).wait()

            pltpu.make_async_copy(edges_hbm.at[tb + k], ib[j], isem[j]).wait()
            pltpu.async_copy(z_hbm.at[ib[j].at[0]], rb[rj], gsem[rj])

            @pl.when(k >= 1)
            def _():
                pltpu.make_async_copy(z_hbm.at[ib[ij].at[0]], rb[pj],
                                      gsem[pj]).wait()
                pltpu.async_copy(rb[pj], acc_sh.at[ib[ij].at[1]], ssem[pj],
                                 add=True)

            @pl.when(k + PF < KCH)
            def _():
                idx_load(k + PF, (j + PF) % NIB)
        return carry

    lax.fori_loop(0, KCH // NIB, group, 0)

    lastj = (KCH - 1) % NIB
    lastr = (KCH - 1) % NRB
    pltpu.make_async_copy(z_hbm.at[ib[lastj].at[0]], rb[lastr],
                          gsem[lastr]).wait()
    pltpu.async_copy(rb[lastr], acc_sh.at[ib[lastj].at[1]], ssem[lastr],
                     add=True)
    for r in range(NRB):
        pltpu.make_async_copy(rb[r], acc_sh.at[ib[r].at[1]], ssem[r]).wait()
    plsc.subcore_barrier()
    pltpu.sync_copy(acc_sh.at[pl.ds(r0, RPT)], out_hbm.at[c, pl.ds(r0, RPT)])


_mp_kernel = functools.partial(
    pl.kernel,
    out_type=jax.ShapeDtypeStruct((2, NP, D), jnp.float32),
    mesh=plsc.VectorSubcoreMesh(**_MESH),
    scratch_types=(
        [pltpu.VMEM((2, CH), jnp.int32) for _ in range(NIB)]
        + [pltpu.VMEM((CH, D), jnp.float32) for _ in range(NRB)]
        + [pltpu.VMEM_SHARED((NP, D), jnp.float32)]
        + [pltpu.SemaphoreType.DMA for _ in range(NIB + 2 * NRB)]
    ),
)(_mp_body)


def _deg_body(edges_hbm, e1_hbm, zdeg_hbm, out_hbm, *scr):
    ib = scr[0:NIB]
    e1_v = scr[NIB]
    acc_sh = scr[NIB + 1]
    isem = scr[NIB + 2:NIB + 2 + NIB]
    ssem = scr[NIB + 2 + NIB:]
    c = lax.axis_index("c")
    s = lax.axis_index("s")
    r0 = s * RPT
    pltpu.sync_copy(zdeg_hbm, acc_sh.at[pl.ds(r0, RPT)])
    pltpu.sync_copy(e1_hbm, e1_v)
    plsc.subcore_barrier()
    tb = (c * 16 + s) * KCH

    def idx_load(k, j):
        pltpu.async_copy(edges_hbm.at[tb + k], ib[j], isem[j])

    for j in range(PF):
        idx_load(j, j)

    def group(g, carry):
        for j in range(NIB):
            k = g * NIB + j
            rj = j % NSS

            @pl.when(k >= NSS)
            def _():
                pltpu.make_async_copy(
                    e1_v, acc_sh.at[ib[(j - NSS) % NIB].at[1]],
                    ssem[rj]).wait()

            pltpu.make_async_copy(edges_hbm.at[tb + k], ib[j], isem[j]).wait()
            pltpu.async_copy(e1_v, acc_sh.at[ib[j].at[1]], ssem[rj], add=True)

            @pl.when(k + PF < KCH)
            def _():
                idx_load(k + PF, (j + PF) % NIB)
        return carry

    lax.fori_loop(0, KCH // NIB, group, 0)
    for r in range(NSS):
        pltpu.make_async_copy(e1_v, acc_sh.at[ib[r].at[1]], ssem[r]).wait()
    plsc.subcore_barrier()
    pltpu.sync_copy(acc_sh.at[pl.ds(r0, RPT)], out_hbm.at[c, pl.ds(r0, RPT)])


_deg_kernel = functools.partial(
    pl.kernel,
    out_type=jax.ShapeDtypeStruct((2, NP, DW), jnp.float32),
    mesh=plsc.VectorSubcoreMesh(**_MESH),
    scratch_types=(
        [pltpu.VMEM((2, CH), jnp.int32) for _ in range(NIB)]
        + [pltpu.VMEM((CH, DW), jnp.float32)]
        + [pltpu.VMEM_SHARED((NP, DW), jnp.float32)]
        + [pltpu.SemaphoreType.DMA for _ in range(NIB + NSS)]
    ),
)(_deg_body)


_R = 1024


def _dinv_of(degp):
    return lax.rsqrt(jnp.sum(degp, axis=(0, 2)) + 1.0)[:, None]


def _zscale_body(x_ref, w_ref, degp_ref, z_ref):
    dinv = _dinv_of(degp_ref[...])
    z_ref[...] = jnp.dot(x_ref[...], w_ref[...],
                         preferred_element_type=jnp.float32) * dinv


_zscale = pl.pallas_call(
    _zscale_body,
    grid=(NP // _R,),
    in_specs=[
        pl.BlockSpec((_R, D), lambda i: (i, 0)),
        pl.BlockSpec((D, D), lambda i: (0, 0)),
        pl.BlockSpec((2, _R, DW), lambda i: (0, i, 0)),
    ],
    out_specs=pl.BlockSpec((_R, D), lambda i: (i, 0)),
    out_shape=jax.ShapeDtypeStruct((NP, D), jnp.float32),
)


def _layer2_body(s_ref, z1_ref, degp_ref, b1_ref, w2_ref, z2_ref):
    dinv = _dinv_of(degp_ref[...])
    s = s_ref[...]
    t = (s[0] + s[1] + z1_ref[...]) * dinv + b1_ref[...]
    h = jnp.maximum(t, 0.0)
    z2_ref[...] = jnp.dot(h, w2_ref[...],
                          preferred_element_type=jnp.float32) * dinv


_layer2 = pl.pallas_call(
    _layer2_body,
    grid=(NP // _R,),
    in_specs=[
        pl.BlockSpec((2, _R, D), lambda i: (0, i, 0)),
        pl.BlockSpec((_R, D), lambda i: (i, 0)),
        pl.BlockSpec((2, _R, DW), lambda i: (0, i, 0)),
        pl.BlockSpec((1, D), lambda i: (0, 0)),
        pl.BlockSpec((D, D), lambda i: (0, 0)),
    ],
    out_specs=pl.BlockSpec((_R, D), lambda i: (i, 0)),
    out_shape=jax.ShapeDtypeStruct((NP, D), jnp.float32),
)


def _head_body(s_ref, z2_ref, degp_ref, b2_ref, wl_ref, bl_ref,
               emb_ref, out_ref):
    dinv = _dinv_of(degp_ref[...])
    s = s_ref[...]
    emb = (s[0] + s[1] + z2_ref[...]) * dinv + b2_ref[...]
    emb_ref[...] = emb
    logits = jnp.dot(emb, wl_ref[...],
                     preferred_element_type=jnp.float32) + bl_ref[...]
    mask = lax.broadcasted_iota(jnp.int32, logits.shape, 1) < OUT
    lm = jnp.where(mask, logits, jnp.float32(-1e30))
    m = jnp.max(lm, axis=1, keepdims=True)
    ex = jnp.where(mask, jnp.exp(logits - m), 0.0)
    lse = jnp.log(jnp.sum(ex, axis=1, keepdims=True))
    out_ref[...] = logits - m - lse


_head = pl.pallas_call(
    _head_body,
    grid=(NP // _R,),
    in_specs=[
        pl.BlockSpec((2, _R, D), lambda i: (0, i, 0)),
        pl.BlockSpec((_R, D), lambda i: (i, 0)),
        pl.BlockSpec((2, _R, DW), lambda i: (0, i, 0)),
        pl.BlockSpec((1, D), lambda i: (0, 0)),
        pl.BlockSpec((D, D), lambda i: (0, 0)),
        pl.BlockSpec((1, D), lambda i: (0, 0)),
    ],
    out_specs=[
        pl.BlockSpec((_R, D), lambda i: (i, 0)),
        pl.BlockSpec((_R, D), lambda i: (i, 0)),
    ],
    out_shape=[
        jax.ShapeDtypeStruct((NP, D), jnp.float32),
        jax.ShapeDtypeStruct((NP, D), jnp.float32),
    ],
)


@jax.jit
def kernel(x, edge_index, W1, b1, W2, b2, Wl, bl):
    pad = EP - E
    srcp = jnp.concatenate([edge_index[0], jnp.zeros((pad,), jnp.int32)])
    dstp = jnp.concatenate([edge_index[1], jnp.full((pad,), N, jnp.int32)])
    edges = jnp.stack(
        [srcp.reshape(NCHUNK, CH), dstp.reshape(NCHUNK, CH)], axis=1)
    x_p = jnp.pad(x, ((0, NP - N), (0, 0)))

    e1 = jnp.zeros((CH, DW), jnp.float32).at[:, 0].set(1.0)
    zdeg = jnp.zeros((RPT, DW), jnp.float32)
    zrow = jnp.zeros((RPT, D), jnp.float32)

    degp = _deg_kernel(edges, e1, zdeg)
    z1 = _zscale(x_p, W1, degp)
    s1 = _mp_kernel(z1, edges, zrow)
    z2 = _layer2(s1, z1, degp, b1.reshape(1, D), W2)
    s2 = _mp_kernel(z2, edges, zrow)
    wl_p = jnp.pad(Wl, ((0, 0), (0, D - OUT)))
    bl_p = jnp.pad(bl, (0, D - OUT)).reshape(1, D)
    emb, outp = _head(s2, z2, degp, b2.reshape(1, D), wl_p, bl_p)
    return (outp[:N, :OUT], emb[:N])


# speedup vs baseline: 1.0620x; 1.0620x over previous
import functools

import jax
import jax.numpy as jnp
from jax import lax
from jax.experimental import pallas as pl
from jax.experimental.pallas import tpu as pltpu
from jax.experimental.pallas import tpu_sc as plsc

N = 10000
D = 128
OUT = 40
NP = 10240
RPT = NP // 16
E = 320000
CH = 128
KCH = 80
EPT = CH * KCH
EP = EPT * 32
NCHUNK = EP // CH
DW = 128
K0 = 128
K1 = 2 * KCH - K0

_MESH = dict(core_axis_name="c", subcore_axis_name="s")

NIB = 8
NRB = 2
PF = 4
NSS = 4


def _mp_body(z_hbm, edges_hbm, zrow_hbm, out_hbm, *scr):
    ib = scr[0:NIB]
    rb = scr[NIB:NIB + NRB]
    acc_sh = scr[NIB + NRB]
    isem = scr[NIB + NRB + 1:NIB + NRB + 1 + NIB]
    gsem = scr[NIB + NRB + 1 + NIB:NIB + NRB + 1 + NIB + NRB]
    ssem = scr[NIB + NRB + 1 + NIB + NRB:]
    c = lax.axis_index("c")
    s = lax.axis_index("s")
    r0 = s * RPT
    pltpu.sync_copy(zrow_hbm, acc_sh.at[pl.ds(r0, RPT)])
    plsc.subcore_barrier()
    tb = lax.select(c == 0, s * K0, 16 * K0 + s * K1)
    kch = lax.select(c == 0, K0, K1)
    ng = lax.select(c == 0, K0 // NIB, K1 // NIB)

    def idx_load(k, j):
        pltpu.async_copy(edges_hbm.at[tb + k], ib[j], isem[j])

    for j in range(PF):
        idx_load(j, j)

    def group(g, carry):
        for j in range(NIB):
            k = g * NIB + j
            rj = j % NRB
            pj = (j - 1) % NRB
            ij = (j - 1) % NIB

            @pl.when(k >= NRB)
            def _():
                pltpu.make_async_copy(
                    rb[rj], acc_sh.at[ib[(j - NRB) % NIB].at[1]],
                    ssem[rj]).wait()

            pltpu.make_async_copy(edges_hbm.at[tb + k], ib[j], isem[j]).wait()
            pltpu.async_copy(z_hbm.at[ib[j].at[0]], rb[rj], gsem[rj])

            @pl.when(k >= 1)
            def _():
                pltpu.make_async_copy(z_hbm.at[ib[ij].at[0]], rb[pj],
                                      gsem[pj]).wait()
                pltpu.async_copy(rb[pj], acc_sh.at[ib[ij].at[1]], ssem[pj],
                                 add=True)

            @pl.when(k + PF < kch)
            def _():
                idx_load(k + PF, (j + PF) % NIB)
        return carry

    lax.fori_loop(0, ng, group, 0)

    lastj = NIB - 1
    lastr = lastj % NRB
    pltpu.make_async_copy(z_hbm.at[ib[lastj].at[0]], rb[lastr],
                          gsem[lastr]).wait()
    pltpu.async_copy(rb[lastr], acc_sh.at[ib[lastj].at[1]], ssem[lastr],
                     add=True)
    for r in range(NRB):
        pltpu.make_async_copy(rb[r], acc_sh.at[ib[r].at[1]], ssem[r]).wait()
    plsc.subcore_barrier()
    pltpu.sync_copy(acc_sh.at[pl.ds(r0, RPT)], out_hbm.at[c, pl.ds(r0, RPT)])


_mp_kernel = functools.partial(
    pl.kernel,
    out_type=jax.ShapeDtypeStruct((2, NP, D), jnp.float32),
    mesh=plsc.VectorSubcoreMesh(**_MESH),
    scratch_types=(
        [pltpu.VMEM((2, CH), jnp.int32) for _ in range(NIB)]
        + [pltpu.VMEM((CH, D), jnp.float32) for _ in range(NRB)]
        + [pltpu.VMEM_SHARED((NP, D), jnp.float32)]
        + [pltpu.SemaphoreType.DMA for _ in range(NIB + 2 * NRB)]
    ),
)(_mp_body)


def _deg_body(edges_hbm, e1_hbm, zdeg_hbm, out_hbm, *scr):
    ib = scr[0:NIB]
    e1_v = scr[NIB]
    acc_sh = scr[NIB + 1]
    isem = scr[NIB + 2:NIB + 2 + NIB]
    ssem = scr[NIB + 2 + NIB:]
    c = lax.axis_index("c")
    s = lax.axis_index("s")
    r0 = s * RPT
    pltpu.sync_copy(zdeg_hbm, acc_sh.at[pl.ds(r0, RPT)])
    pltpu.sync_copy(e1_hbm, e1_v)
    plsc.subcore_barrier()
    tb = (c * 16 + s) * KCH

    def idx_load(k, j):
        pltpu.async_copy(edges_hbm.at[tb + k], ib[j], isem[j])

    for j in range(PF):
        idx_load(j, j)

    def group(g, carry):
        for j in range(NIB):
            k = g * NIB + j
            rj = j % NSS

            @pl.when(k >= NSS)
            def _():
                pltpu.make_async_copy(
                    e1_v, acc_sh.at[ib[(j - NSS) % NIB].at[1]],
                    ssem[rj]).wait()

            pltpu.make_async_copy(edges_hbm.at[tb + k], ib[j], isem[j]).wait()
            pltpu.async_copy(e1_v, acc_sh.at[ib[j].at[1]], ssem[rj], add=True)

            @pl.when(k + PF < KCH)
            def _():
                idx_load(k + PF, (j + PF) % NIB)
        return carry

    lax.fori_loop(0, KCH // NIB, group, 0)
    for r in range(NSS):
        pltpu.make_async_copy(e1_v, acc_sh.at[ib[r].at[1]], ssem[r]).wait()
    plsc.subcore_barrier()
    pltpu.sync_copy(acc_sh.at[pl.ds(r0, RPT)], out_hbm.at[c, pl.ds(r0, RPT)])


_deg_kernel = functools.partial(
    pl.kernel,
    out_type=jax.ShapeDtypeStruct((2, NP, DW), jnp.float32),
    mesh=plsc.VectorSubcoreMesh(**_MESH),
    scratch_types=(
        [pltpu.VMEM((2, CH), jnp.int32) for _ in range(NIB)]
        + [pltpu.VMEM((CH, DW), jnp.float32)]
        + [pltpu.VMEM_SHARED((NP, DW), jnp.float32)]
        + [pltpu.SemaphoreType.DMA for _ in range(NIB + NSS)]
    ),
)(_deg_body)


_R = 1024


def _dinv_of(degp):
    return lax.rsqrt(jnp.sum(degp, axis=(0, 2)) + 1.0)[:, None]


def _zscale_body(x_ref, w_ref, degp_ref, z_ref):
    dinv = _dinv_of(degp_ref[...])
    z_ref[...] = jnp.dot(x_ref[...], w_ref[...],
                         preferred_element_type=jnp.float32) * dinv


_zscale = pl.pallas_call(
    _zscale_body,
    grid=(NP // _R,),
    in_specs=[
        pl.BlockSpec((_R, D), lambda i: (i, 0)),
        pl.BlockSpec((D, D), lambda i: (0, 0)),
        pl.BlockSpec((2, _R, DW), lambda i: (0, i, 0)),
    ],
    out_specs=pl.BlockSpec((_R, D), lambda i: (i, 0)),
    out_shape=jax.ShapeDtypeStruct((NP, D), jnp.float32),
)


def _layer2_body(s_ref, z1_ref, degp_ref, b1_ref, w2_ref, z2_ref):
    dinv = _dinv_of(degp_ref[...])
    s = s_ref[...]
    t = (s[0] + s[1] + z1_ref[...]) * dinv + b1_ref[...]
    h = jnp.maximum(t, 0.0)
    z2_ref[...] = jnp.dot(h, w2_ref[...],
                          preferred_element_type=jnp.float32) * dinv


_layer2 = pl.pallas_call(
    _layer2_body,
    grid=(NP // _R,),
    in_specs=[
        pl.BlockSpec((2, _R, D), lambda i: (0, i, 0)),
        pl.BlockSpec((_R, D), lambda i: (i, 0)),
        pl.BlockSpec((2, _R, DW), lambda i: (0, i, 0)),
        pl.BlockSpec((1, D), lambda i: (0, 0)),
        pl.BlockSpec((D, D), lambda i: (0, 0)),
    ],
    out_specs=pl.BlockSpec((_R, D), lambda i: (i, 0)),
    out_shape=jax.ShapeDtypeStruct((NP, D), jnp.float32),
)


def _head_body(s_ref, z2_ref, degp_ref, b2_ref, wl_ref, bl_ref,
               emb_ref, out_ref):
    dinv = _dinv_of(degp_ref[...])
    s = s_ref[...]
    emb = (s[0] + s[1] + z2_ref[...]) * dinv + b2_ref[...]
    emb_ref[...] = emb
    logits = jnp.dot(emb, wl_ref[...],
                     preferred_element_type=jnp.float32) + bl_ref[...]
    mask = lax.broadcasted_iota(jnp.int32, logits.shape, 1) < OUT
    lm = jnp.where(mask, logits, jnp.float32(-1e30))
    m = jnp.max(lm, axis=1, keepdims=True)
    ex = jnp.where(mask, jnp.exp(logits - m), 0.0)
    lse = jnp.log(jnp.sum(ex, axis=1, keepdims=True))
    out_ref[...] = logits - m - lse


_head = pl.pallas_call(
    _head_body,
    grid=(NP // _R,),
    in_specs=[
        pl.BlockSpec((2, _R, D), lambda i: (0, i, 0)),
        pl.BlockSpec((_R, D), lambda i: (i, 0)),
        pl.BlockSpec((2, _R, DW), lambda i: (0, i, 0)),
        pl.BlockSpec((1, D), lambda i: (0, 0)),
        pl.BlockSpec((D, D), lambda i: (0, 0)),
        pl.BlockSpec((1, D), lambda i: (0, 0)),
    ],
    out_specs=[
        pl.BlockSpec((_R, D), lambda i: (i, 0)),
        pl.BlockSpec((_R, D), lambda i: (i, 0)),
    ],
    out_shape=[
        jax.ShapeDtypeStruct((NP, D), jnp.float32),
        jax.ShapeDtypeStruct((NP, D), jnp.float32),
    ],
)


@jax.jit
def kernel(x, edge_index, W1, b1, W2, b2, Wl, bl):
    pad = EP - E
    srcp = jnp.concatenate([edge_index[0], jnp.zeros((pad,), jnp.int32)])
    dstp = jnp.concatenate([edge_index[1], jnp.full((pad,), N, jnp.int32)])
    edges = jnp.stack(
        [srcp.reshape(NCHUNK, CH), dstp.reshape(NCHUNK, CH)], axis=1)
    x_p = jnp.pad(x, ((0, NP - N), (0, 0)))

    e1 = jnp.zeros((CH, DW), jnp.float32).at[:, 0].set(1.0)
    zdeg = jnp.zeros((RPT, DW), jnp.float32)
    zrow = jnp.zeros((RPT, D), jnp.float32)

    degp = _deg_kernel(edges, e1, zdeg)
    z1 = _zscale(x_p, W1, degp)
    s1 = _mp_kernel(z1, edges, zrow)
    z2 = _layer2(s1, z1, degp, b1.reshape(1, D), W2)
    s2 = _mp_kernel(z2, edges, zrow)
    wl_p = jnp.pad(Wl, ((0, 0), (0, D - OUT)))
    bl_p = jnp.pad(bl, (0, D - OUT)).reshape(1, D)
    emb, outp = _head(s2, z2, degp, b2.reshape(1, D), wl_p, bl_p)
    return (outp[:N, :OUT], emb[:N])
